# Initial kernel scaffold; baseline (speedup 1.0000x reference)
import jax
import jax.numpy as jnp
from jax.experimental import pallas as pl


def kernel(x, edge_index, batch, W1, b1, W2, b2):
    raise NotImplementedError("write your pallas kernel here")


# speedup vs baseline: 10.8353x; 10.8353x over previous
import functools

import jax
import jax.numpy as jnp
from jax import lax
from jax.experimental import pallas as pl
from jax.experimental.pallas import tpu as pltpu
from jax.experimental.pallas import tpu_sc as plsc

N = 10000
D = 128
E = 320000
NC = 2
NS = 16
K = 128
NW = NC * NS
CPT = -(-E // (NW * K))
E_PAD = NW * K * CPT
TRASH = N
NACC = 10240
ROWS_PER_TILE = NACC // NS

_mesh = plsc.VectorSubcoreMesh(core_axis_name="c", subcore_axis_name="s")


def _deg_body(dst_hbm, out_hbm, didx, ones_v, zbuf, deg_sh):
    c = lax.axis_index("c")
    s = lax.axis_index("s")
    zv = jnp.zeros((16,), jnp.float32)
    ov = jnp.ones((16,), jnp.float32)
    for j in range(K // 16):
        ones_v[pl.ds(j * 16, 16)] = ov

    def zb(i, carry):
        zbuf[pl.ds(i * 16, 16)] = zv
        return carry

    lax.fori_loop(0, ROWS_PER_TILE // 16, zb, 0)
    pltpu.sync_copy(zbuf, deg_sh.at[pl.ds(s * ROWS_PER_TILE, ROWS_PER_TILE)])
    plsc.subcore_barrier()

    base = (c * NS + s) * CPT

    def body(k, carry):
        e0 = (base + k) * K
        pltpu.sync_copy(dst_hbm.at[pl.ds(e0, K)], didx)
        pltpu.sync_copy(ones_v, deg_sh.at[didx], add=True)
        return carry

    lax.fori_loop(0, CPT, body, 0)
    plsc.subcore_barrier()
    pltpu.sync_copy(deg_sh.at[pl.ds(s * ROWS_PER_TILE, ROWS_PER_TILE)],
                    out_hbm.at[c, pl.ds(s * ROWS_PER_TILE, ROWS_PER_TILE)])


_deg_call = functools.partial(
    pl.kernel,
    out_type=jax.ShapeDtypeStruct((NC, NACC), jnp.float32),
    mesh=_mesh,
    scratch_types=[
        pltpu.VMEM((K,), jnp.int32),
        pltpu.VMEM((K,), jnp.float32),
        pltpu.VMEM((ROWS_PER_TILE,), jnp.float32),
        pltpu.VMEM_SHARED((NACC,), jnp.float32),
    ],
)(_deg_body)


def _edge_body(y_hbm, src_hbm, dst_hbm, out_hbm, sidx, didx, rows, acc_sh, sem):
    c = lax.axis_index("c")
    s = lax.axis_index("s")
    zv = jnp.zeros((16,), jnp.float32)

    def zero_rows(r, carry):
        for j in range(D // 16):
            rows[r, pl.ds(j * 16, 16)] = zv
        return carry

    lax.fori_loop(0, K, zero_rows, 0)
    for i in range(ROWS_PER_TILE // K):
        pltpu.sync_copy(rows, acc_sh.at[pl.ds(s * ROWS_PER_TILE + i * K, K)])
    plsc.subcore_barrier()

    base = (c * NS + s) * CPT

    def body(k, carry):
        e0 = (base + k) * K
        pltpu.sync_copy(src_hbm.at[pl.ds(e0, K)], sidx)
        pltpu.sync_copy(dst_hbm.at[pl.ds(e0, K)], didx)
        pltpu.async_copy(y_hbm.at[sidx], rows, sem).wait()
        pltpu.sync_copy(rows, acc_sh.at[didx], add=True)
        return carry

    lax.fori_loop(0, CPT, body, 0)
    plsc.subcore_barrier()
    pltpu.sync_copy(acc_sh.at[pl.ds(s * ROWS_PER_TILE, ROWS_PER_TILE)],
                    out_hbm.at[c, pl.ds(s * ROWS_PER_TILE, ROWS_PER_TILE)])


_edge_call = functools.partial(
    pl.kernel,
    out_type=jax.ShapeDtypeStruct((NC, NACC, D), jnp.float32),
    mesh=_mesh,
    scratch_types=[
        pltpu.VMEM((K,), jnp.int32),
        pltpu.VMEM((K,), jnp.int32),
        pltpu.VMEM((K, D), jnp.float32),
        pltpu.VMEM_SHARED((NACC, D), jnp.float32),
        pltpu.SemaphoreType.DMA,
    ],
)(_edge_body)


R = 1000


def _dinv(d0, d1):
    return lax.rsqrt(jnp.maximum(d0 + d1 + 1.0, 1e-12))


def _mm_scale_body(x_ref, w_ref, d0_ref, d1_ref, o_ref):
    d = _dinv(d0_ref[...], d1_ref[...])
    o_ref[...] = jnp.dot(x_ref[...], w_ref[...],
                         preferred_element_type=jnp.float32) * d


def _fuse_body(a0_ref, a1_ref, y1_ref, d0_ref, d1_ref, w_ref, b_ref, o_ref):
    d = _dinv(d0_ref[...], d1_ref[...])
    h = d * (a0_ref[...] + a1_ref[...] + y1_ref[...]) + b_ref[...]
    h = jnp.maximum(h, 0.0)
    o_ref[...] = jnp.dot(h, w_ref[...],
                         preferred_element_type=jnp.float32) * d


def _final_body(a0_ref, a1_ref, y2_ref, d0_ref, d1_ref, b_ref, o_ref):
    d = _dinv(d0_ref[...], d1_ref[...])
    o_ref[...] = d * (a0_ref[...] + a1_ref[...] + y2_ref[...]) + b_ref[...]


_row_spec = pl.BlockSpec((R, D), lambda i: (i, 0))
_deg_spec = pl.BlockSpec((R, 1), lambda i: (i, 0))
_full_spec = pl.BlockSpec((D, D), lambda i: (0, 0))
_bias_spec = pl.BlockSpec((1, D), lambda i: (0, 0))
_out_struct = jax.ShapeDtypeStruct((N, D), jnp.float32)

_mm_scale = pl.pallas_call(
    _mm_scale_body,
    grid=(N // R,),
    in_specs=[_row_spec, _full_spec, _deg_spec, _deg_spec],
    out_specs=_row_spec,
    out_shape=_out_struct,
)

_fuse = pl.pallas_call(
    _fuse_body,
    grid=(N // R,),
    in_specs=[_row_spec, _row_spec, _row_spec, _deg_spec, _deg_spec,
              _full_spec, _bias_spec],
    out_specs=_row_spec,
    out_shape=_out_struct,
)

_final = pl.pallas_call(
    _final_body,
    grid=(N // R,),
    in_specs=[_row_spec, _row_spec, _row_spec, _deg_spec, _deg_spec,
              _bias_spec],
    out_specs=_row_spec,
    out_shape=_out_struct,
)


def kernel(x, edge_index, batch, W1, b1, W2, b2):
    src = edge_index[0].astype(jnp.int32)
    dst = edge_index[1].astype(jnp.int32)
    pad = E_PAD - E
    src_p = jnp.concatenate([src, jnp.zeros((pad,), jnp.int32)])
    dst_p = jnp.concatenate([dst, jnp.full((pad,), TRASH, jnp.int32)])

    deg_part = _deg_call(dst_p)
    deg0 = deg_part[0, :N].reshape(N, 1)
    deg1 = deg_part[1, :N].reshape(N, 1)

    y1 = _mm_scale(x, W1, deg0, deg1)
    acc1 = _edge_call(y1, src_p, dst_p)
    y2 = _fuse(acc1[0, :N], acc1[1, :N], y1, deg0, deg1, W2, b1.reshape(1, D))
    acc2 = _edge_call(y2, src_p, dst_p)
    out = _final(acc2[0, :N], acc2[1, :N], y2, deg0, deg1, b2.reshape(1, D))
    return (out, batch)
